# baseline (speedup 1.0000x reference)
import jax
import jax.numpy as jnp
from jax import lax
from jax.experimental import pallas as pl
from jax.experimental.pallas import tpu as pltpu
from jax.experimental.pallas import tpu_sc as plsc

_N = 10000
_E = 320000
_R = 4
_H = 256
_NH = 3
_FF = 4 * _H
_EPS = 1e-5

_NC = 2
_NS = 16
_NW = _NC * _NS
_EB = 80
_EPW = _E // _NW
_NIT = _EPW // _EB
_EPT = _E // _NS
_NIT2 = _EPT // _EB
_GRP = 4
_CHUNK = 128

_BN = 1000
_NB = _N // _BN


def _make_sc():
    out_type = [jax.ShapeDtypeStruct((_NC * _N, _CHUNK), jnp.float32)]
    scratch = (
        [pltpu.VMEM_SHARED((_N, _CHUNK), jnp.float32)]
        + [pltpu.VMEM((2, _EB), jnp.int32) for _ in range(_GRP)]
        + [pltpu.VMEM((_EB, _CHUNK), jnp.float32) for _ in range(_GRP)]
        + [pltpu.SemaphoreType.DMA] * (2 * _GRP)
    )
    mesh = plsc.VectorSubcoreMesh(core_axis_name="c", subcore_axis_name="s",
                                  num_cores=_NC, num_subcores=_NS)

    def body(t0, t1, pidx, zeros128, part, acc, *rest):
        iv = rest[:_GRP]
        rows = rest[_GRP:2 * _GRP]
        isem = rest[2 * _GRP:3 * _GRP]
        gsem = rest[3 * _GRP:4 * _GRP]
        c = lax.axis_index("c")
        s = lax.axis_index("s")
        row0 = s * _NIT2

        def issue_idx(i, k):
            pltpu.async_copy(pidx.at[row0 + i], iv[k], isem[k])

        def wait_idx(k):
            pltpu.make_async_copy(pidx.at[row0], iv[k], isem[k]).wait()

        def issue_gather(i, k):
            @pl.when(c == 0)
            def _():
                pltpu.async_copy(t0.at[iv[k].at[0]], rows[k], gsem[k])

            @pl.when(c != 0)
            def _():
                pltpu.async_copy(t1.at[iv[k].at[0]], rows[k], gsem[k])

        def wait_gather(k):
            @pl.when(c == 0)
            def _():
                pltpu.make_async_copy(t0.at[iv[k].at[0]], rows[k],
                                      gsem[k]).wait()

            @pl.when(c != 0)
            def _():
                pltpu.make_async_copy(t1.at[iv[k].at[0]], rows[k],
                                      gsem[k]).wait()

        @pl.when(s < _NB)
        def _():
            pltpu.sync_copy(zeros128.at[pl.ds(s * _BN, _BN)],
                            acc.at[pl.ds(s * _BN, _BN)])

        plsc.subcore_barrier()

        def iteration(i, k, do_gather=True, do_prefetch=True):
            if do_gather:
                wait_idx((k + 1) % _GRP)
                issue_gather(i + 1, (k + 1) % _GRP)
            wait_gather(k)
            pltpu.sync_copy(rows[k], acc.at[iv[k].at[1]], add=True)
            if do_prefetch:
                issue_idx(i + 3, (k + 3) % _GRP)

        for k in range(3):
            issue_idx(k, k)
        wait_idx(0)
        issue_gather(0, 0)
        iteration(0, 0)

        ngrp = (_NIT2 - 5) // _GRP

        def group(j, carry):
            for t in range(_GRP):
                iteration(j * _GRP + 1 + t, (1 + t) % _GRP)
            return carry

        lax.fori_loop(0, ngrp, group, 0)

        for i in range(1 + ngrp * _GRP, _NIT2):
            iteration(i, i % _GRP, do_gather=(i + 1 < _NIT2),
                      do_prefetch=(i + 3 < _NIT2))

        plsc.subcore_barrier()

        @pl.when(s < _NB)
        def _():
            pltpu.sync_copy(acc.at[pl.ds(s * _BN, _BN)],
                            part.at[pl.ds(c * _N + s * _BN, _BN)])

    return pl.kernel(body, out_type=out_type, mesh=mesh,
                     scratch_types=scratch)


def _make_deg():
    out_type = [jax.ShapeDtypeStruct((_NC * _N, _CHUNK), jnp.float32)]
    scratch = (
        [pltpu.VMEM_SHARED((_N, _CHUNK), jnp.float32)]
        + [pltpu.VMEM((_EB,), jnp.int32) for _ in range(_GRP)]
        + [pltpu.VMEM((_EB, _CHUNK), jnp.float32)]
        + [pltpu.SemaphoreType.DMA] * (2 * _GRP)
    )
    mesh = plsc.VectorSubcoreMesh(core_axis_name="c", subcore_axis_name="s",
                                  num_cores=_NC, num_subcores=_NS)

    def body(didx2, zeros128, ones, part, acc, *rest):
        ivd = rest[:_GRP]
        ones_v = rest[_GRP]
        isem = rest[_GRP + 1:_GRP + 1 + _GRP]
        ssem = rest[_GRP + 1 + _GRP:]
        c = lax.axis_index("c")
        s = lax.axis_index("s")
        wid = c * _NS + s
        row0 = wid * _NIT

        def issue_idx(i, k):
            pltpu.async_copy(didx2.at[row0 + i], ivd[k], isem[k])

        def wait_idx(k):
            pltpu.make_async_copy(didx2.at[row0], ivd[k], isem[k]).wait()

        @pl.when(s < _NB)
        def _():
            pltpu.sync_copy(zeros128.at[pl.ds(s * _BN, _BN)],
                            acc.at[pl.ds(s * _BN, _BN)])

        pltpu.sync_copy(ones, ones_v)
        plsc.subcore_barrier()

        def iteration(i, k, first=False, do_prefetch=True):
            wait_idx(k)
            pltpu.sync_copy(ones_v, acc.at[ivd[k]], add=True)
            if do_prefetch:
                issue_idx(i + 3, (k + 3) % _GRP)

        for k in range(3):
            issue_idx(k, k)
        iteration(0, 0, first=True)

        def group(j, carry):
            for t in range(_GRP):
                iteration(j * _GRP + 1 + t, (1 + t) % _GRP)
            return carry

        lax.fori_loop(0, (_NIT - 1) // _GRP - 1, group, 0)

        for t in range(_GRP):
            i = _NIT - _GRP + t
            iteration(i, i % _GRP, do_prefetch=(i + 3 < _NIT))

        plsc.subcore_barrier()

        @pl.when(s < _NB)
        def _():
            pltpu.sync_copy(acc.at[pl.ds(s * _BN, _BN)],
                            part.at[pl.ds(c * _N + s * _BN, _BN)])

    return pl.kernel(body, out_type=out_type, mesh=mesh,
                     scratch_types=scratch)


_sc_cache = {}


def _get_sc(kind):
    if kind not in _sc_cache:
        _sc_cache[kind] = _make_deg() if kind == "deg" else _make_sc()
    return _sc_cache[kind]


def _tables_body(x_ref, w_ref, o0_ref, o1_ref):
    t = jnp.dot(x_ref[...], w_ref[0], preferred_element_type=jnp.float32)
    o0_ref[...] = t[:, :_CHUNK]
    o1_ref[...] = t[:, _CHUNK:]


def _tables(x, w_rel):
    di = x.shape[1]
    return pl.pallas_call(
        _tables_body,
        grid=(_R, _NB),
        in_specs=[
            pl.BlockSpec((_BN, di), lambda r, n: (n, 0)),
            pl.BlockSpec((1, di, _H), lambda r, n: (r, 0, 0)),
        ],
        out_specs=[
            pl.BlockSpec((_BN, _CHUNK), lambda r, n: (r * _NB + n, 0)),
            pl.BlockSpec((_BN, _CHUNK), lambda r, n: (r * _NB + n, 0)),
        ],
        out_shape=[jax.ShapeDtypeStruct((_R * _N, _CHUNK), jnp.float32)] * 2,
    )(x, w_rel)


def _combine1_body(x_ref, wr_ref, p0_ref, p1_ref,
                   dga_ref, dgb_ref, b_ref, h_ref, s_ref, q_ref):
    agg = jnp.concatenate([p0_ref[...], p1_ref[...]], axis=1)
    deg = jnp.maximum(dga_ref[...][:, 0:1] + dgb_ref[...][:, 0:1], 1.0)
    h = jnp.dot(x_ref[...], wr_ref[...], preferred_element_type=jnp.float32)
    h = h + agg / deg + b_ref[...]
    h_ref[...] = h
    s_ref[...] = jnp.sum(h, axis=0).reshape(1, 1, _H)
    q_ref[...] = jnp.sum(h * h, axis=0).reshape(1, 1, _H)


def _combine1(x, w_root, part, degp, b):
    di = x.shape[1]
    return pl.pallas_call(
        _combine1_body,
        grid=(_NB,),
        in_specs=[
            pl.BlockSpec((_BN, di), lambda n: (n, 0)),
            pl.BlockSpec((di, _H), lambda n: (0, 0)),
            pl.BlockSpec((_BN, _CHUNK), lambda n: (n, 0)),
            pl.BlockSpec((_BN, _CHUNK), lambda n: (_NB + n, 0)),
            pl.BlockSpec((_BN, _CHUNK), lambda n: (n, 0)),
            pl.BlockSpec((_BN, _CHUNK), lambda n: (_NB + n, 0)),
            pl.BlockSpec((1, _H), lambda n: (0, 0)),
        ],
        out_specs=[
            pl.BlockSpec((_BN, _H), lambda n: (n, 0)),
            pl.BlockSpec((1, 1, _H), lambda n: (n, 0, 0)),
            pl.BlockSpec((1, 1, _H), lambda n: (n, 0, 0)),
        ],
        out_shape=[
            jax.ShapeDtypeStruct((_N, _H), jnp.float32),
            jax.ShapeDtypeStruct((_NB, 1, _H), jnp.float32),
            jax.ShapeDtypeStruct((_NB, 1, _H), jnp.float32),
        ],
    )(x, w_root, part, part, degp, degp, b)


def _combine2_body(h_ref, s_ref, q_ref, g_ref, be_ref, o_ref):
    mu = jnp.sum(s_ref[...].reshape(_NB, _H), axis=0, keepdims=True) * (1.0 / _N)
    msq = jnp.sum(q_ref[...].reshape(_NB, _H), axis=0, keepdims=True) * (1.0 / _N)
    var = msq - mu * mu
    rstd = lax.rsqrt(var + _EPS)
    hn = (h_ref[...] - mu) * rstd * g_ref[...] + be_ref[...]
    o_ref[...] = jnp.maximum(hn, 0.0)


def _combine2(h, ssum, ssq, g, be):
    return pl.pallas_call(
        _combine2_body,
        grid=(_NB,),
        in_specs=[
            pl.BlockSpec((_BN, _H), lambda n: (n, 0)),
            pl.BlockSpec((_NB, 1, _H), lambda n: (0, 0, 0)),
            pl.BlockSpec((_NB, 1, _H), lambda n: (0, 0, 0)),
            pl.BlockSpec((1, _H), lambda n: (0, 0)),
            pl.BlockSpec((1, _H), lambda n: (0, 0)),
        ],
        out_specs=pl.BlockSpec((_BN, _H), lambda n: (n, 0)),
        out_shape=jax.ShapeDtypeStruct((_N, _H), jnp.float32),
    )(h, ssum, ssq, g, be)


def _heads_body(h_ref, w1_ref, b1_ref, w2_ref, b2_ref, o_ref):
    h = h_ref[...]
    w1 = w1_ref[...]
    b1 = b1_ref[...]
    w2 = w2_ref[...]
    b2 = b2_ref[...]
    cols = []
    for r in range(_NH):
        t = jnp.dot(h, w1[r], preferred_element_type=jnp.float32) + b1[r][None, :]
        t = jnp.maximum(t, 0.0)
        col = jnp.sum(t * w2[r, :, 0][None, :], axis=1, keepdims=True) + b2[r, 0]
        cols.append(col)
    o_ref[...] = jnp.concatenate(cols, axis=1)


def _heads(h, hw1, hb1, hw2, hb2):
    return pl.pallas_call(
        _heads_body,
        grid=(_NB,),
        in_specs=[
            pl.BlockSpec((_BN, _H), lambda n: (n, 0)),
            pl.BlockSpec((_NH, _H, _FF), lambda n: (0, 0, 0)),
            pl.BlockSpec((_NH, _FF), lambda n: (0, 0)),
            pl.BlockSpec((_NH, _FF, 1), lambda n: (0, 0, 0)),
            pl.BlockSpec((_NH, 1), lambda n: (0, 0)),
        ],
        out_specs=pl.BlockSpec((_BN, _NH), lambda n: (n, 0)),
        out_shape=jax.ShapeDtypeStruct((_N, _NH), jnp.float32),
    )(h, hw1, hb1, hw2, hb2)


def kernel(x, edge_index, edge_type, W_rel1, W_root1, b1, g1, be1,
           W_rel2, W_root2, b2, g2, be2, W_rel3, W_root3, b3, g3, be3,
           hw1, hb1, hw2, hb2):
    src = edge_index[0].astype(jnp.int32)
    dst = edge_index[1].astype(jnp.int32)
    et = edge_type.astype(jnp.int32)
    gidx = et * _N + src
    didx = dst
    gidx2 = gidx.reshape(_NW * _NIT, _EB)
    didx2 = didx.reshape(_NW * _NIT, _EB)
    pidx = jnp.stack([gidx2, didx2], axis=1)
    zeros128 = jnp.zeros((_N, _CHUNK), jnp.float32)
    ones = jnp.ones((_EB, _CHUNK), jnp.float32)

    layers = [
        (W_rel1, W_root1, b1, g1, be1),
        (W_rel2, W_root2, b2, g2, be2),
        (W_rel3, W_root3, b3, g3, be3),
    ]
    h = x
    (degp,) = _get_sc("deg")(didx2, zeros128, ones)
    for li, (w_rel, w_root, b, g, be) in enumerate(layers):
        t0, t1 = _tables(h, w_rel)
        (part,) = _get_sc("aggr")(t0, t1, pidx, zeros128)
        hmid, ssum, ssq = _combine1(h, w_root, part, degp,
                                    b.reshape(1, _H))
        h = _combine2(hmid, ssum, ssq, g.reshape(1, _H), be.reshape(1, _H))
    return _heads(h, hw1, hb1, hw2, hb2)


# speedup vs baseline: 1.0087x; 1.0087x over previous
import jax
import jax.numpy as jnp
from jax import lax
from jax.experimental import pallas as pl
from jax.experimental.pallas import tpu as pltpu
from jax.experimental.pallas import tpu_sc as plsc

_N = 10000
_E = 320000
_R = 4
_H = 256
_NH = 3
_FF = 4 * _H
_EPS = 1e-5

_NC = 2
_NS = 16
_NW = _NC * _NS
_EB = 80
_EPW = _E // _NW
_NIT = _EPW // _EB
_EPT = _E // _NS
_NIT2 = _EPT // _EB
_GRP = 4
_CHUNK = 128

_BN = 1000
_NB = _N // _BN


def _make_sc():
    out_type = [jax.ShapeDtypeStruct((_NC * _N, _CHUNK), jnp.float32)]
    scratch = (
        [pltpu.VMEM_SHARED((_N, _CHUNK), jnp.float32)]
        + [pltpu.VMEM((_EB,), jnp.int32) for _ in range(2 * _GRP)]
        + [pltpu.VMEM((_EB, _CHUNK), jnp.float32) for _ in range(_GRP)]
        + [pltpu.SemaphoreType.DMA] * (2 * _GRP)
    )
    mesh = plsc.VectorSubcoreMesh(core_axis_name="c", subcore_axis_name="s",
                                  num_cores=_NC, num_subcores=_NS)

    def body(t0, t1, gidx2, didx2, zeros128, part, acc, *rest):
        ivg = rest[:_GRP]
        ivd = rest[_GRP:2 * _GRP]
        rows = rest[2 * _GRP:3 * _GRP]
        isem = rest[3 * _GRP:4 * _GRP]
        gsem = rest[4 * _GRP:5 * _GRP]
        c = lax.axis_index("c")
        s = lax.axis_index("s")
        row0 = s * _NIT2

        def issue_idx(i, k):
            pltpu.async_copy(gidx2.at[row0 + i], ivg[k], isem[k])
            pltpu.async_copy(didx2.at[row0 + i], ivd[k], isem[k])

        def wait_idx(k):
            pltpu.make_async_copy(gidx2.at[row0], ivg[k], isem[k]).wait()
            pltpu.make_async_copy(didx2.at[row0], ivd[k], isem[k]).wait()

        def issue_gather(i, k):
            @pl.when(c == 0)
            def _():
                pltpu.async_copy(t0.at[ivg[k]], rows[k], gsem[k])

            @pl.when(c != 0)
            def _():
                pltpu.async_copy(t1.at[ivg[k]], rows[k], gsem[k])

        def wait_gather(k):
            @pl.when(c == 0)
            def _():
                pltpu.make_async_copy(t0.at[ivg[k]], rows[k], gsem[k]).wait()

            @pl.when(c != 0)
            def _():
                pltpu.make_async_copy(t1.at[ivg[k]], rows[k], gsem[k]).wait()

        @pl.when(s < _NB)
        def _():
            pltpu.sync_copy(zeros128.at[pl.ds(s * _BN, _BN)],
                            acc.at[pl.ds(s * _BN, _BN)])

        plsc.subcore_barrier()

        def iteration(i, k, do_gather=True, do_prefetch=True):
            if do_gather:
                wait_idx((k + 1) % _GRP)
                issue_gather(i + 1, (k + 1) % _GRP)
            wait_gather(k)
            pltpu.sync_copy(rows[k], acc.at[ivd[k]], add=True)
            if do_prefetch:
                issue_idx(i + 3, (k + 3) % _GRP)

        for k in range(3):
            issue_idx(k, k)
        wait_idx(0)
        issue_gather(0, 0)
        iteration(0, 0)

        ngrp = (_NIT2 - 5) // _GRP

        def group(j, carry):
            for t in range(_GRP):
                iteration(j * _GRP + 1 + t, (1 + t) % _GRP)
            return carry

        lax.fori_loop(0, ngrp, group, 0)

        for i in range(1 + ngrp * _GRP, _NIT2):
            iteration(i, i % _GRP, do_gather=(i + 1 < _NIT2),
                      do_prefetch=(i + 3 < _NIT2))

        plsc.subcore_barrier()

        @pl.when(s < _NB)
        def _():
            pltpu.sync_copy(acc.at[pl.ds(s * _BN, _BN)],
                            part.at[pl.ds(c * _N + s * _BN, _BN)])

    return pl.kernel(body, out_type=out_type, mesh=mesh,
                     scratch_types=scratch)


def _make_deg():
    out_type = [jax.ShapeDtypeStruct((_NC * _N, _CHUNK), jnp.float32)]
    scratch = (
        [pltpu.VMEM_SHARED((_N, _CHUNK), jnp.float32)]
        + [pltpu.VMEM((_EB,), jnp.int32) for _ in range(_GRP)]
        + [pltpu.VMEM((_EB, _CHUNK), jnp.float32)]
        + [pltpu.SemaphoreType.DMA] * (2 * _GRP)
    )
    mesh = plsc.VectorSubcoreMesh(core_axis_name="c", subcore_axis_name="s",
                                  num_cores=_NC, num_subcores=_NS)

    def body(didx2, zeros128, ones, part, acc, *rest):
        ivd = rest[:_GRP]
        ones_v = rest[_GRP]
        isem = rest[_GRP + 1:_GRP + 1 + _GRP]
        ssem = rest[_GRP + 1 + _GRP:]
        c = lax.axis_index("c")
        s = lax.axis_index("s")
        wid = c * _NS + s
        row0 = wid * _NIT

        def issue_idx(i, k):
            pltpu.async_copy(didx2.at[row0 + i], ivd[k], isem[k])

        def wait_idx(k):
            pltpu.make_async_copy(didx2.at[row0], ivd[k], isem[k]).wait()

        @pl.when(s < _NB)
        def _():
            pltpu.sync_copy(zeros128.at[pl.ds(s * _BN, _BN)],
                            acc.at[pl.ds(s * _BN, _BN)])

        pltpu.sync_copy(ones, ones_v)
        plsc.subcore_barrier()

        def iteration(i, k, first=False, do_prefetch=True):
            wait_idx(k)
            pltpu.sync_copy(ones_v, acc.at[ivd[k]], add=True)
            if do_prefetch:
                issue_idx(i + 3, (k + 3) % _GRP)

        for k in range(3):
            issue_idx(k, k)
        iteration(0, 0, first=True)

        def group(j, carry):
            for t in range(_GRP):
                iteration(j * _GRP + 1 + t, (1 + t) % _GRP)
            return carry

        lax.fori_loop(0, (_NIT - 1) // _GRP - 1, group, 0)

        for t in range(_GRP):
            i = _NIT - _GRP + t
            iteration(i, i % _GRP, do_prefetch=(i + 3 < _NIT))

        plsc.subcore_barrier()

        @pl.when(s < _NB)
        def _():
            pltpu.sync_copy(acc.at[pl.ds(s * _BN, _BN)],
                            part.at[pl.ds(c * _N + s * _BN, _BN)])

    return pl.kernel(body, out_type=out_type, mesh=mesh,
                     scratch_types=scratch)


_sc_cache = {}


def _get_sc(kind):
    if kind not in _sc_cache:
        _sc_cache[kind] = _make_deg() if kind == "deg" else _make_sc()
    return _sc_cache[kind]


def _tables_body(x_ref, w_ref, o0_ref, o1_ref):
    t = jnp.dot(x_ref[...], w_ref[0], preferred_element_type=jnp.float32)
    o0_ref[...] = t[:, :_CHUNK]
    o1_ref[...] = t[:, _CHUNK:]


def _tables(x, w_rel):
    di = x.shape[1]
    return pl.pallas_call(
        _tables_body,
        grid=(_R, _NB),
        in_specs=[
            pl.BlockSpec((_BN, di), lambda r, n: (n, 0)),
            pl.BlockSpec((1, di, _H), lambda r, n: (r, 0, 0)),
        ],
        out_specs=[
            pl.BlockSpec((_BN, _CHUNK), lambda r, n: (r * _NB + n, 0)),
            pl.BlockSpec((_BN, _CHUNK), lambda r, n: (r * _NB + n, 0)),
        ],
        out_shape=[jax.ShapeDtypeStruct((_R * _N, _CHUNK), jnp.float32)] * 2,
    )(x, w_rel)


def _combine1_body(x_ref, wr_ref, p0_ref, p1_ref,
                   dga_ref, dgb_ref, b_ref, h_ref, s_ref, q_ref):
    agg = jnp.concatenate([p0_ref[...], p1_ref[...]], axis=1)
    deg = jnp.maximum(dga_ref[...][:, 0:1] + dgb_ref[...][:, 0:1], 1.0)
    h = jnp.dot(x_ref[...], wr_ref[...], preferred_element_type=jnp.float32)
    h = h + agg / deg + b_ref[...]
    h_ref[...] = h
    s_ref[...] = jnp.sum(h, axis=0).reshape(1, 1, _H)
    q_ref[...] = jnp.sum(h * h, axis=0).reshape(1, 1, _H)


def _combine1(x, w_root, part, degp, b):
    di = x.shape[1]
    return pl.pallas_call(
        _combine1_body,
        grid=(_NB,),
        in_specs=[
            pl.BlockSpec((_BN, di), lambda n: (n, 0)),
            pl.BlockSpec((di, _H), lambda n: (0, 0)),
            pl.BlockSpec((_BN, _CHUNK), lambda n: (n, 0)),
            pl.BlockSpec((_BN, _CHUNK), lambda n: (_NB + n, 0)),
            pl.BlockSpec((_BN, _CHUNK), lambda n: (n, 0)),
            pl.BlockSpec((_BN, _CHUNK), lambda n: (_NB + n, 0)),
            pl.BlockSpec((1, _H), lambda n: (0, 0)),
        ],
        out_specs=[
            pl.BlockSpec((_BN, _H), lambda n: (n, 0)),
            pl.BlockSpec((1, 1, _H), lambda n: (n, 0, 0)),
            pl.BlockSpec((1, 1, _H), lambda n: (n, 0, 0)),
        ],
        out_shape=[
            jax.ShapeDtypeStruct((_N, _H), jnp.float32),
            jax.ShapeDtypeStruct((_NB, 1, _H), jnp.float32),
            jax.ShapeDtypeStruct((_NB, 1, _H), jnp.float32),
        ],
    )(x, w_root, part, part, degp, degp, b)


def _combine2_body(h_ref, s_ref, q_ref, g_ref, be_ref, o_ref):
    mu = jnp.sum(s_ref[...].reshape(_NB, _H), axis=0, keepdims=True) * (1.0 / _N)
    msq = jnp.sum(q_ref[...].reshape(_NB, _H), axis=0, keepdims=True) * (1.0 / _N)
    var = msq - mu * mu
    rstd = lax.rsqrt(var + _EPS)
    hn = (h_ref[...] - mu) * rstd * g_ref[...] + be_ref[...]
    o_ref[...] = jnp.maximum(hn, 0.0)


def _combine2(h, ssum, ssq, g, be):
    return pl.pallas_call(
        _combine2_body,
        grid=(_NB,),
        in_specs=[
            pl.BlockSpec((_BN, _H), lambda n: (n, 0)),
            pl.BlockSpec((_NB, 1, _H), lambda n: (0, 0, 0)),
            pl.BlockSpec((_NB, 1, _H), lambda n: (0, 0, 0)),
            pl.BlockSpec((1, _H), lambda n: (0, 0)),
            pl.BlockSpec((1, _H), lambda n: (0, 0)),
        ],
        out_specs=pl.BlockSpec((_BN, _H), lambda n: (n, 0)),
        out_shape=jax.ShapeDtypeStruct((_N, _H), jnp.float32),
    )(h, ssum, ssq, g, be)


def _heads_body(h_ref, w1_ref, b1_ref, w2_ref, b2_ref, o_ref):
    h = h_ref[...]
    w1 = w1_ref[...]
    b1 = b1_ref[...]
    w2 = w2_ref[...]
    b2 = b2_ref[...]
    cols = []
    for r in range(_NH):
        t = jnp.dot(h, w1[r], preferred_element_type=jnp.float32) + b1[r][None, :]
        t = jnp.maximum(t, 0.0)
        col = jnp.sum(t * w2[r, :, 0][None, :], axis=1, keepdims=True) + b2[r, 0]
        cols.append(col)
    o_ref[...] = jnp.concatenate(cols, axis=1)


def _heads(h, hw1, hb1, hw2, hb2):
    return pl.pallas_call(
        _heads_body,
        grid=(_NB,),
        in_specs=[
            pl.BlockSpec((_BN, _H), lambda n: (n, 0)),
            pl.BlockSpec((_NH, _H, _FF), lambda n: (0, 0, 0)),
            pl.BlockSpec((_NH, _FF), lambda n: (0, 0)),
            pl.BlockSpec((_NH, _FF, 1), lambda n: (0, 0, 0)),
            pl.BlockSpec((_NH, 1), lambda n: (0, 0)),
        ],
        out_specs=pl.BlockSpec((_BN, _NH), lambda n: (n, 0)),
        out_shape=jax.ShapeDtypeStruct((_N, _NH), jnp.float32),
    )(h, hw1, hb1, hw2, hb2)


def kernel(x, edge_index, edge_type, W_rel1, W_root1, b1, g1, be1,
           W_rel2, W_root2, b2, g2, be2, W_rel3, W_root3, b3, g3, be3,
           hw1, hb1, hw2, hb2):
    src = edge_index[0].astype(jnp.int32)
    dst = edge_index[1].astype(jnp.int32)
    et = edge_type.astype(jnp.int32)
    gidx = et * _N + src
    didx = dst
    gidx2 = gidx.reshape(_NW * _NIT, _EB)
    didx2 = didx.reshape(_NW * _NIT, _EB)
    zeros128 = jnp.zeros((_N, _CHUNK), jnp.float32)
    ones = jnp.ones((_EB, _CHUNK), jnp.float32)

    layers = [
        (W_rel1, W_root1, b1, g1, be1),
        (W_rel2, W_root2, b2, g2, be2),
        (W_rel3, W_root3, b3, g3, be3),
    ]
    h = x
    (degp,) = _get_sc("deg")(didx2, zeros128, ones)
    for li, (w_rel, w_root, b, g, be) in enumerate(layers):
        t0, t1 = _tables(h, w_rel)
        (part,) = _get_sc("aggr")(t0, t1, gidx2, didx2, zeros128)
        hmid, ssum, ssq = _combine1(h, w_root, part, degp,
                                    b.reshape(1, _H))
        h = _combine2(hmid, ssum, ssq, g.reshape(1, _H), be.reshape(1, _H))
    return _heads(h, hw1, hb1, hw2, hb2)
